# Initial kernel scaffold; baseline (speedup 1.0000x reference)
import jax
import jax.numpy as jnp
from jax.experimental import pallas as pl


def kernel(node_features, mesh_edge_features, world_edge_features, mesh_senders, mesh_receivers, world_senders, world_receivers, params):
    raise NotImplementedError("write your pallas kernel here")


# speedup vs baseline: 1.1057x; 1.1057x over previous
import functools

import jax
import jax.numpy as jnp
from jax.experimental import pallas as pl
from jax.experimental.pallas import tpu as pltpu

LATENT = 128


def _ln(h, gamma, beta):
    mu = jnp.mean(h, axis=-1, keepdims=True)
    var = jnp.mean((h - mu) ** 2, axis=-1, keepdims=True)
    return (h - mu) * jax.lax.rsqrt(var + 1e-5) * gamma + beta


def _enc_body(x_ref, w1_ref, b1_ref, w2_ref, b2_ref, g_ref, bt_ref, o_ref):
    h = jnp.maximum(
        jnp.dot(x_ref[...], w1_ref[...], preferred_element_type=jnp.float32)
        + b1_ref[...], 0.0)
    h = jnp.dot(h, w2_ref[...], preferred_element_type=jnp.float32) + b2_ref[...]
    o_ref[...] = _ln(h, g_ref[...], bt_ref[...])


def _edge_body(e_ref, g_ref, w1a_ref, b1_ref, w2_ref, b2_ref, gam_ref, bt_ref,
               o_ref):
    e = e_ref[...]
    h = jnp.maximum(
        jnp.dot(e, w1a_ref[...], preferred_element_type=jnp.float32)
        + g_ref[...] + b1_ref[...], 0.0)
    h = jnp.dot(h, w2_ref[...], preferred_element_type=jnp.float32) + b2_ref[...]
    o_ref[...] = e + _ln(h, gam_ref[...], bt_ref[...])


def _node_body(x_ref, am_ref, aw_ref, w1a_ref, w1b_ref, w1c_ref, b1_ref,
               w2_ref, b2_ref, gam_ref, bt_ref, o_ref):
    x = x_ref[...]
    h = (jnp.dot(x, w1a_ref[...], preferred_element_type=jnp.float32)
         + jnp.dot(am_ref[...], w1b_ref[...], preferred_element_type=jnp.float32)
         + jnp.dot(aw_ref[...], w1c_ref[...], preferred_element_type=jnp.float32)
         + b1_ref[...])
    h = jnp.maximum(h, 0.0)
    h = jnp.dot(h, w2_ref[...], preferred_element_type=jnp.float32) + b2_ref[...]
    o_ref[...] = x + _ln(h, gam_ref[...], bt_ref[...])


def _proj_body(x_ref, w_ref, o_ref):
    o_ref[...] = jnp.dot(x_ref[...], w_ref[...],
                         preferred_element_type=jnp.float32)


def _dec_body(x_ref, w1_ref, b1_ref, w2_ref, b2_ref, o_ref):
    h = jnp.maximum(
        jnp.dot(x_ref[...], w1_ref[...], preferred_element_type=jnp.float32)
        + b1_ref[...], 0.0)
    o_ref[...] = (jnp.dot(h, w2_ref[...], preferred_element_type=jnp.float32)
                  + b2_ref[...])


def _param_spec(shape):
    return pl.BlockSpec(shape, lambda i: (0,) * len(shape))


def _rows_spec(blk, ncols):
    return pl.BlockSpec((blk, ncols), lambda i: (i, 0))


def _call_rows(body, n_rows, blk, row_args, row_widths, param_args, out_cols,
               interpret=False):
    grid = n_rows // blk
    in_specs = ([_rows_spec(blk, w) for w in row_widths]
                + [_param_spec(p.shape) for p in param_args])
    return pl.pallas_call(
        body,
        grid=(grid,),
        in_specs=in_specs,
        out_specs=_rows_spec(blk, out_cols),
        out_shape=jax.ShapeDtypeStruct((n_rows, out_cols), jnp.float32),
        interpret=interpret,
    )(*row_args, *param_args)


def kernel(node_features, mesh_edge_features, world_edge_features,
           mesh_senders, mesh_receivers, world_senders, world_receivers,
           params, *, interpret=False):
    n = node_features.shape[0]
    em = mesh_edge_features.shape[0]
    ew = world_edge_features.shape[0]

    ms = mesh_senders.astype(jnp.int32)
    mr = mesh_receivers.astype(jnp.int32)
    ws = world_senders.astype(jnp.int32)
    wr = world_receivers.astype(jnp.int32)

    f32 = jnp.float32

    def pad_cols(x, to):
        return jnp.pad(x, ((0, 0), (0, to - x.shape[1])))

    def fold_norm(p, mean, std):
        w1 = p["W1"] / std[:, None]
        b1 = p["b1"] - (mean / std) @ p["W1"]
        return w1, b1

    BN = 2000

    def enc(x, w1, b1, p):
        din = x.shape[1]
        dpad = 16 if din <= 16 else 128
        xp = pad_cols(x.astype(f32), dpad)
        w1p = jnp.pad(w1, ((0, dpad - din), (0, 0)))
        pa = [w1p, b1.reshape(1, -1), p["W2"], p["b2"].reshape(1, -1),
              p["g"].reshape(1, -1), p["beta"].reshape(1, -1)]
        return _call_rows(_enc_body, x.shape[0], BN, [xp], [dpad], pa, LATENT,
                          interpret=interpret)

    pe = params["enc_node"]
    node = enc(node_features, pe["W1"], pe["b1"], pe)
    pm_ = params["enc_mesh"]
    w1m, b1m = fold_norm(pm_, params["mesh_norm_mean"], params["mesh_norm_std"])
    me = enc(mesh_edge_features, w1m, b1m, pm_)
    pw_ = params["enc_world"]
    w1w, b1w = fold_norm(pw_, params["world_norm_mean"], params["world_norm_std"])
    we = enc(world_edge_features, w1w, b1w, pw_)

    steps = params["proc_mesh"]["W1"].shape[0]
    BE = 4000

    def split_w1(w1):
        return w1[:LATENT], w1[LATENT:2 * LATENT], w1[2 * LATENT:]

    for i in range(steps):
        pmesh = jax.tree_util.tree_map(lambda a: a[i], params["proc_mesh"])
        pworld = jax.tree_util.tree_map(lambda a: a[i], params["proc_world"])
        pnode = jax.tree_util.tree_map(lambda a: a[i], params["proc_node"])

        w1a_m, w1b_m, w1c_m = split_w1(pmesh["W1"])
        w1a_w, w1b_w, w1c_w = split_w1(pworld["W1"])
        w1a_n, w1b_n, w1c_n = split_w1(pnode["W1"])

        wstack = jnp.concatenate([w1b_m, w1c_m, w1b_w, w1c_w], axis=1)
        z = _call_rows(_proj_body, n, BN, [node], [LATENT], [wstack],
                       4 * LATENT, interpret=interpret)
        zms, zmr, zws, zwr = (z[:, :128], z[:, 128:256], z[:, 256:384],
                              z[:, 384:])

        g_m = zms[ms] + zmr[mr]
        g_w = zws[ws] + zwr[wr]

        pa_m = [w1a_m, pmesh["b1"].reshape(1, -1), pmesh["W2"],
                pmesh["b2"].reshape(1, -1), pmesh["g"].reshape(1, -1),
                pmesh["beta"].reshape(1, -1)]
        me = _call_rows(_edge_body, em, BE, [me, g_m], [LATENT, LATENT], pa_m,
                        LATENT, interpret=interpret)
        pa_w = [w1a_w, pworld["b1"].reshape(1, -1), pworld["W2"],
                pworld["b2"].reshape(1, -1), pworld["g"].reshape(1, -1),
                pworld["beta"].reshape(1, -1)]
        we = _call_rows(_edge_body, ew, BN, [we, g_w], [LATENT, LATENT], pa_w,
                        LATENT, interpret=interpret)

        agg_m = jax.ops.segment_sum(me, mr, num_segments=n)
        agg_w = jax.ops.segment_sum(we, wr, num_segments=n)

        pa_n = [w1a_n, w1b_n, w1c_n, pnode["b1"].reshape(1, -1), pnode["W2"],
                pnode["b2"].reshape(1, -1), pnode["g"].reshape(1, -1),
                pnode["beta"].reshape(1, -1)]
        node = _call_rows(_node_body, n, BN, [node, agg_m, agg_w],
                          [LATENT] * 3, pa_n, LATENT, interpret=interpret)

    pd = params["dec"]
    out_dim = pd["W2"].shape[1]
    w2p = pad_cols(pd["W2"], LATENT)
    b2p = jnp.pad(pd["b2"], (0, LATENT - out_dim))
    pa_d = [pd["W1"], pd["b1"].reshape(1, -1), w2p, b2p.reshape(1, -1)]
    out = _call_rows(_dec_body, n, BN, [node], [LATENT], pa_d, LATENT,
                     interpret=interpret)
    return out[:, :out_dim]


# speedup vs baseline: 4.5506x; 4.1156x over previous
import functools

import jax
import jax.numpy as jnp
from jax import lax
from jax.experimental import pallas as pl
from jax.experimental.pallas import tpu as pltpu
from jax.experimental.pallas import tpu_sc as plsc

LATENT = 128
_CHUNK = 128
_NC, _NS = 2, 16
_NW = _NC * _NS
_N_ACC = 10240


def _ln(h, gamma, beta):
    mu = jnp.mean(h, axis=-1, keepdims=True)
    var = jnp.mean((h - mu) ** 2, axis=-1, keepdims=True)
    return (h - mu) * jax.lax.rsqrt(var + 1e-5) * gamma + beta


def _enc_body(x_ref, w1_ref, b1_ref, w2_ref, b2_ref, g_ref, bt_ref, o_ref):
    h = jnp.maximum(
        jnp.dot(x_ref[...], w1_ref[...], preferred_element_type=jnp.float32)
        + b1_ref[...], 0.0)
    h = jnp.dot(h, w2_ref[...], preferred_element_type=jnp.float32) + b2_ref[...]
    o_ref[...] = _ln(h, g_ref[...], bt_ref[...])


def _edge_body(e_ref, g_ref, w1a_ref, b1_ref, w2_ref, b2_ref, gam_ref, bt_ref,
               wsel_ref, o_ref, p_ref):
    e = e_ref[...]
    h = jnp.maximum(
        jnp.dot(e, w1a_ref[0], preferred_element_type=jnp.float32)
        + g_ref[...] + b1_ref[0], 0.0)
    h = jnp.dot(h, w2_ref[0], preferred_element_type=jnp.float32) + b2_ref[0]
    e2 = e + _ln(h, gam_ref[0], bt_ref[0])
    o_ref[...] = e2
    p_ref[...] = jnp.dot(e2, wsel_ref[0], preferred_element_type=jnp.float32)


def _node_body(x_ref, a0_ref, a1_ref, w1a_ref, b1_ref, w2_ref, b2_ref,
               gam_ref, bt_ref, o_ref):
    x = x_ref[...]
    h = (jnp.dot(x, w1a_ref[...], preferred_element_type=jnp.float32)
         + a0_ref[...] + a1_ref[...] + b1_ref[...])
    h = jnp.maximum(h, 0.0)
    h = jnp.dot(h, w2_ref[...], preferred_element_type=jnp.float32) + b2_ref[...]
    o_ref[...] = x + _ln(h, gam_ref[...], bt_ref[...])


def _proj_body(x_ref, w_ref, os_ref, or_ref):
    z = jnp.dot(x_ref[...], w_ref[...], preferred_element_type=jnp.float32)
    os_ref[0] = z[:, :LATENT]
    or_ref[0] = z[:, LATENT:2 * LATENT]
    os_ref[1] = z[:, 2 * LATENT:3 * LATENT]
    or_ref[1] = z[:, 3 * LATENT:]


def _dec_body(x_ref, w1_ref, b1_ref, w2_ref, b2_ref, o_ref):
    h = jnp.maximum(
        jnp.dot(x_ref[...], w1_ref[...], preferred_element_type=jnp.float32)
        + b1_ref[...], 0.0)
    o_ref[...] = (jnp.dot(h, w2_ref[...], preferred_element_type=jnp.float32)
                  + b2_ref[...])


def _param_spec(shape):
    return pl.BlockSpec(shape, lambda i: (0,) * len(shape))


def _rows_spec(blk, ncols):
    return pl.BlockSpec((blk, ncols), lambda i: (i, 0))


def _call_rows(body, n_rows, blk, row_args, row_widths, param_args, out_cols,
               interpret=False):
    grid = n_rows // blk
    in_specs = ([_rows_spec(blk, w) for w in row_widths]
                + [_param_spec(p.shape) for p in param_args])
    return pl.pallas_call(
        body,
        grid=(grid,),
        in_specs=in_specs,
        out_specs=_rows_spec(blk, out_cols),
        out_shape=jax.ShapeDtypeStruct((n_rows, out_cols), jnp.float32),
        interpret=interpret,
    )(*row_args, *param_args)


def _call_proj(node, wstack, n, blk, interpret=False):
    grid = n // blk
    spec = pl.BlockSpec((2, blk, LATENT), lambda i: (0, i, 0))
    return pl.pallas_call(
        _proj_body,
        grid=(grid,),
        in_specs=[_rows_spec(blk, LATENT), _param_spec(wstack.shape)],
        out_specs=[spec, spec],
        out_shape=[jax.ShapeDtypeStruct((2, n, LATENT), jnp.float32)] * 2,
        interpret=interpret,
    )(node, wstack)


def _call_edges(e, g, stacked, e_tot, blk, mesh_blocks, interpret=False):
    grid = e_tot // blk

    def psel(shape):
        return pl.BlockSpec((1,) + shape[1:],
                            lambda i: (jnp.where(i < mesh_blocks, 0, 1),)
                            + (0,) * (len(shape) - 1))

    in_specs = ([_rows_spec(blk, LATENT)] * 2
                + [psel(p.shape) for p in stacked])
    out = pl.pallas_call(
        _edge_body,
        grid=(grid,),
        in_specs=in_specs,
        out_specs=[_rows_spec(blk, LATENT)] * 2,
        out_shape=[jax.ShapeDtypeStruct((e_tot, LATENT), jnp.float32)] * 2,
        interpret=interpret,
    )(e, g, *stacked)
    return out


def _sc_mesh():
    return plsc.VectorSubcoreMesh(core_axis_name="c", subcore_axis_name="s",
                                  num_cores=_NC, num_subcores=_NS)


def _idx_spec():
    return pl.BlockSpec((1, _CHUNK), lambda i: (0, i))


def _row_blk_spec():
    return pl.BlockSpec((_CHUNK, LATENT), lambda i: (i, 0))


def _sc_gather_sum(zs, zr, i_s, i_r, e_tot):
    f32 = jnp.float32

    @functools.partial(
        pl.kernel,
        out_type=jax.ShapeDtypeStruct((e_tot, LATENT), f32),
        mesh=_sc_mesh(),
        scratch_types=[pltpu.VMEM((_CHUNK, LATENT), f32),
                       pltpu.SemaphoreType.DMA,
                       pltpu.SemaphoreType.DMA],
    )
    def k(zs_h, zr_h, is_h, ir_h, g_h, tmp, sem1, sem2):
        def body(is_ref, ir_ref, o_ref):
            c1 = pltpu.async_copy(zs_h.at[is_ref.at[0]], o_ref, sem1)
            c2 = pltpu.async_copy(zr_h.at[ir_ref.at[0]], tmp, sem2)
            c1.wait()
            c2.wait()

            @pl.loop(0, _CHUNK)
            def _(r):
                for c in range(0, LATENT, 16):
                    sl = pl.ds(c, 16)
                    o_ref.at[r, sl][...] = (o_ref.at[r, sl][...]
                                            + tmp.at[r, sl][...])

        pltpu.emit_pipeline(
            body,
            grid=(e_tot // _CHUNK,),
            in_specs=[_idx_spec(), _idx_spec()],
            out_specs=[_row_blk_spec()],
            core_axis_name=("c", "s"),
            dimension_semantics=(pltpu.PARALLEL,),
        )(is_h, ir_h, g_h)

    return k(zs, zr, i_s, i_r)


def _sc_segment_sum(p, idx, e_tot):
    f32 = jnp.float32
    per_tile = _N_ACC // _NS

    @functools.partial(
        pl.kernel,
        out_type=jax.ShapeDtypeStruct((_NC, _N_ACC, LATENT), f32),
        mesh=_sc_mesh(),
        scratch_types=[pltpu.VMEM_SHARED((_N_ACC, LATENT), f32)],
    )
    def k(p_h, i_h, z_h, o_h, agg):
        cidx = lax.axis_index("c")
        sidx = lax.axis_index("s")
        base = sidx * per_tile

        pltpu.sync_copy(z_h.at[pl.ds(base, per_tile)],
                        agg.at[pl.ds(base, per_tile)])
        plsc.subcore_barrier()

        def body(x_ref, i_ref):
            pltpu.sync_copy(x_ref, agg.at[i_ref.at[0]], add=True)

        pltpu.emit_pipeline(
            body,
            grid=(e_tot // _CHUNK,),
            in_specs=[_row_blk_spec(), _idx_spec()],
            out_specs=[],
            core_axis_name=("c", "s"),
            dimension_semantics=(pltpu.PARALLEL,),
        )(p_h, i_h)

        plsc.subcore_barrier()

        @pl.loop(0, per_tile // _CHUNK)
        def _(j):
            off = base + j * _CHUNK
            pltpu.sync_copy(agg.at[pl.ds(off, _CHUNK)],
                            o_h.at[cidx, pl.ds(off, _CHUNK)])

    return k(p, idx, jnp.zeros((_N_ACC, LATENT), f32))


def kernel(node_features, mesh_edge_features, world_edge_features,
           mesh_senders, mesh_receivers, world_senders, world_receivers,
           params, *, interpret=False):
    n = node_features.shape[0]
    em = mesh_edge_features.shape[0]
    ew = world_edge_features.shape[0]
    span = _CHUNK * _NW
    em_pad = ((em + span - 1) // span) * span
    ew_pad = ((ew + span - 1) // span) * span
    e_tot = em_pad + ew_pad

    f32 = jnp.float32

    def pad_idx(ix, tot, fill):
        ix = ix.astype(jnp.int32)
        return jnp.pad(ix, (0, tot - ix.shape[0]), constant_values=fill)

    i_s = jnp.concatenate([pad_idx(mesh_senders, em_pad, 0),
                           pad_idx(world_senders, ew_pad, 0) + n]
                          ).reshape(1, e_tot)
    i_r = jnp.concatenate([pad_idx(mesh_receivers, em_pad, 0),
                           pad_idx(world_receivers, ew_pad, 0) + n]
                          ).reshape(1, e_tot)
    i_sc = jnp.concatenate([pad_idx(mesh_receivers, em_pad, n),
                            pad_idx(world_receivers, ew_pad, n)]
                           ).reshape(1, e_tot)

    def pad_cols(x, to):
        return jnp.pad(x, ((0, 0), (0, to - x.shape[1])))

    def fold_norm(p, mean, std):
        w1 = p["W1"] / std[:, None]
        b1 = p["b1"] - (mean / std) @ p["W1"]
        return w1, b1

    BN = 2000

    def enc(x, w1, b1, p):
        din = x.shape[1]
        dpad = 16
        xp = pad_cols(x.astype(f32), dpad)
        w1p = jnp.pad(w1, ((0, dpad - din), (0, 0)))
        pa = [w1p, b1.reshape(1, -1), p["W2"], p["b2"].reshape(1, -1),
              p["g"].reshape(1, -1), p["beta"].reshape(1, -1)]
        return _call_rows(_enc_body, x.shape[0], BN, [xp], [dpad], pa, LATENT,
                          interpret=interpret)

    pe = params["enc_node"]
    node = enc(node_features, pe["W1"], pe["b1"], pe)
    pm_ = params["enc_mesh"]
    w1m, b1m = fold_norm(pm_, params["mesh_norm_mean"], params["mesh_norm_std"])
    me = enc(mesh_edge_features, w1m, b1m, pm_)
    pw_ = params["enc_world"]
    w1w, b1w = fold_norm(pw_, params["world_norm_mean"], params["world_norm_std"])
    we = enc(world_edge_features, w1w, b1w, pw_)

    e_lat = jnp.concatenate([jnp.pad(me, ((0, em_pad - em), (0, 0))),
                             jnp.pad(we, ((0, ew_pad - ew), (0, 0)))])

    steps = params["proc_mesh"]["W1"].shape[0]
    BE = 2048
    mesh_blocks = em_pad // BE

    def split_w1(w1):
        return w1[:LATENT], w1[LATENT:2 * LATENT], w1[2 * LATENT:]

    for i in range(steps):
        pmesh = jax.tree_util.tree_map(lambda a: a[i], params["proc_mesh"])
        pworld = jax.tree_util.tree_map(lambda a: a[i], params["proc_world"])
        pnode = jax.tree_util.tree_map(lambda a: a[i], params["proc_node"])

        w1a_m, w1b_m, w1c_m = split_w1(pmesh["W1"])
        w1a_w, w1b_w, w1c_w = split_w1(pworld["W1"])
        w1a_n, w1b_n, w1c_n = split_w1(pnode["W1"])

        wstack = jnp.concatenate([w1b_m, w1c_m, w1b_w, w1c_w], axis=1)
        zs, zr = _call_proj(node, wstack, n, BN, interpret=interpret)

        g = _sc_gather_sum(zs.reshape(2 * n, LATENT),
                           zr.reshape(2 * n, LATENT), i_s, i_r, e_tot)

        stacked = [
            jnp.stack([w1a_m, w1a_w]),
            jnp.stack([pmesh["b1"], pworld["b1"]]).reshape(2, 1, LATENT),
            jnp.stack([pmesh["W2"], pworld["W2"]]),
            jnp.stack([pmesh["b2"], pworld["b2"]]).reshape(2, 1, LATENT),
            jnp.stack([pmesh["g"], pworld["g"]]).reshape(2, 1, LATENT),
            jnp.stack([pmesh["beta"], pworld["beta"]]).reshape(2, 1, LATENT),
            jnp.stack([w1b_n, w1c_n]),
        ]
        e_lat, p_proj = _call_edges(e_lat, g, stacked, e_tot, BE, mesh_blocks,
                                    interpret=interpret)

        part = _sc_segment_sum(p_proj, i_sc, e_tot)

        pa_n = [w1a_n, pnode["b1"].reshape(1, -1), pnode["W2"],
                pnode["b2"].reshape(1, -1), pnode["g"].reshape(1, -1),
                pnode["beta"].reshape(1, -1)]
        node = _call_rows(_node_body, n, BN, [node, part[0], part[1]],
                          [LATENT] * 3, pa_n, LATENT, interpret=interpret)

    pd = params["dec"]
    out_dim = pd["W2"].shape[1]
    w2p = pad_cols(pd["W2"], LATENT)
    b2p = jnp.pad(pd["b2"], (0, LATENT - out_dim))
    pa_d = [pd["W1"], pd["b1"].reshape(1, -1), w2p, b2p.reshape(1, -1)]
    out = _call_rows(_dec_body, n, BN, [node], [LATENT], pa_d, LATENT,
                     interpret=interpret)
    return out[:, :out_dim]


# speedup vs baseline: 7.7414x; 1.7012x over previous
import functools

import jax
import jax.numpy as jnp
from jax import lax
from jax.experimental import pallas as pl
from jax.experimental.pallas import tpu as pltpu
from jax.experimental.pallas import tpu_sc as plsc

LATENT = 128
_CHUNK = 128
_NC, _NS = 2, 16
_NW = _NC * _NS
_N_ACC = 10240


def _ln(h, gamma, beta):
    mu = jnp.mean(h, axis=-1, keepdims=True)
    var = jnp.mean((h - mu) ** 2, axis=-1, keepdims=True)
    return (h - mu) * jax.lax.rsqrt(var + 1e-5) * gamma + beta


def _enc_body(x_ref, w1_ref, b1_ref, w2_ref, b2_ref, g_ref, bt_ref, o_ref):
    h = jnp.maximum(
        jnp.dot(x_ref[...], w1_ref[...], preferred_element_type=jnp.float32)
        + b1_ref[...], 0.0)
    h = jnp.dot(h, w2_ref[...], preferred_element_type=jnp.float32) + b2_ref[...]
    o_ref[...] = _ln(h, g_ref[...], bt_ref[...])


def _edge_body(e_ref, gs_ref, gr_ref, w1a_ref, b1_ref, w2_ref, b2_ref,
               gam_ref, bt_ref, wsel_ref, o_ref, p_ref):
    e = e_ref[...]
    h = jnp.maximum(
        jnp.dot(e, w1a_ref[0], preferred_element_type=jnp.float32)
        + gs_ref[...] + gr_ref[...] + b1_ref[0], 0.0)
    h = jnp.dot(h, w2_ref[0], preferred_element_type=jnp.float32) + b2_ref[0]
    e2 = e + _ln(h, gam_ref[0], bt_ref[0])
    o_ref[...] = e2
    p_ref[...] = jnp.dot(e2, wsel_ref[0], preferred_element_type=jnp.float32)


def _node_body(x_ref, a0_ref, a1_ref, w1a_ref, b1_ref, w2_ref, b2_ref,
               gam_ref, bt_ref, o_ref):
    x = x_ref[...]
    h = (jnp.dot(x, w1a_ref[...], preferred_element_type=jnp.float32)
         + a0_ref[...] + a1_ref[...] + b1_ref[...])
    h = jnp.maximum(h, 0.0)
    h = jnp.dot(h, w2_ref[...], preferred_element_type=jnp.float32) + b2_ref[...]
    o_ref[...] = x + _ln(h, gam_ref[...], bt_ref[...])


def _proj_body(x_ref, w_ref, os_ref, or_ref):
    z = jnp.dot(x_ref[...], w_ref[...], preferred_element_type=jnp.float32)
    os_ref[0] = z[:, :LATENT]
    or_ref[0] = z[:, LATENT:2 * LATENT]
    os_ref[1] = z[:, 2 * LATENT:3 * LATENT]
    or_ref[1] = z[:, 3 * LATENT:]


def _dec_body(x_ref, w1_ref, b1_ref, w2_ref, b2_ref, o_ref):
    h = jnp.maximum(
        jnp.dot(x_ref[...], w1_ref[...], preferred_element_type=jnp.float32)
        + b1_ref[...], 0.0)
    o_ref[...] = (jnp.dot(h, w2_ref[...], preferred_element_type=jnp.float32)
                  + b2_ref[...])


def _param_spec(shape):
    return pl.BlockSpec(shape, lambda i: (0,) * len(shape))


def _rows_spec(blk, ncols):
    return pl.BlockSpec((blk, ncols), lambda i: (i, 0))


def _call_rows(body, n_rows, blk, row_args, row_widths, param_args, out_cols,
               interpret=False):
    grid = n_rows // blk
    in_specs = ([_rows_spec(blk, w) for w in row_widths]
                + [_param_spec(p.shape) for p in param_args])
    return pl.pallas_call(
        body,
        grid=(grid,),
        in_specs=in_specs,
        out_specs=_rows_spec(blk, out_cols),
        out_shape=jax.ShapeDtypeStruct((n_rows, out_cols), jnp.float32),
        interpret=interpret,
    )(*row_args, *param_args)


def _call_proj(node, wstack, n, blk, interpret=False):
    grid = n // blk
    spec = pl.BlockSpec((2, blk, LATENT), lambda i: (0, i, 0))
    return pl.pallas_call(
        _proj_body,
        grid=(grid,),
        in_specs=[_rows_spec(blk, LATENT), _param_spec(wstack.shape)],
        out_specs=[spec, spec],
        out_shape=[jax.ShapeDtypeStruct((2, n, LATENT), jnp.float32)] * 2,
        interpret=interpret,
    )(node, wstack)


def _call_edges(e, gs, gr, stacked, e_tot, blk, mesh_blocks, interpret=False):
    grid = e_tot // blk

    def psel(shape):
        return pl.BlockSpec((1,) + shape[1:],
                            lambda i: (jnp.where(i < mesh_blocks, 0, 1),)
                            + (0,) * (len(shape) - 1))

    in_specs = ([_rows_spec(blk, LATENT)] * 3
                + [psel(p.shape) for p in stacked])
    out = pl.pallas_call(
        _edge_body,
        grid=(grid,),
        in_specs=in_specs,
        out_specs=[_rows_spec(blk, LATENT)] * 2,
        out_shape=[jax.ShapeDtypeStruct((e_tot, LATENT), jnp.float32)] * 2,
        interpret=interpret,
    )(e, gs, gr, *stacked)
    return out


def _sc_mesh():
    return plsc.VectorSubcoreMesh(core_axis_name="c", subcore_axis_name="s",
                                  num_cores=_NC, num_subcores=_NS)


def _idx_spec():
    return pl.BlockSpec((1, _CHUNK), lambda i: (0, i))


def _row_blk_spec():
    return pl.BlockSpec((_CHUNK, LATENT), lambda i: (i, 0))


def _sc_gather_sum(zs, zr, i_s, i_r, e_tot):
    f32 = jnp.float32

    @functools.partial(
        pl.kernel,
        out_type=[jax.ShapeDtypeStruct((e_tot, LATENT), f32),
                  jax.ShapeDtypeStruct((e_tot, LATENT), f32)],
        mesh=_sc_mesh(),
        scratch_types=[pltpu.SemaphoreType.DMA, pltpu.SemaphoreType.DMA],
    )
    def k(zs_h, zr_h, is_h, ir_h, gs_h, gr_h, sem1, sem2):
        def body(is_ref, ir_ref, os_ref, or_ref):
            c1 = pltpu.async_copy(zs_h.at[is_ref.at[0]], os_ref, sem1)
            c2 = pltpu.async_copy(zr_h.at[ir_ref.at[0]], or_ref, sem2)
            c1.wait()
            c2.wait()

        pltpu.emit_pipeline(
            body,
            grid=(e_tot // _CHUNK,),
            in_specs=[_idx_spec(), _idx_spec()],
            out_specs=[_row_blk_spec(), _row_blk_spec()],
            core_axis_name=("c", "s"),
            dimension_semantics=(pltpu.PARALLEL,),
        )(is_h, ir_h, gs_h, gr_h)

    return k(zs, zr, i_s, i_r)


def _sc_segment_sum(p, idx, e_tot):
    f32 = jnp.float32
    per_tile = _N_ACC // _NS

    @functools.partial(
        pl.kernel,
        out_type=jax.ShapeDtypeStruct((_NC, _N_ACC, LATENT), f32),
        mesh=_sc_mesh(),
        scratch_types=[pltpu.VMEM_SHARED((_N_ACC, LATENT), f32)],
    )
    def k(p_h, i_h, z_h, o_h, agg):
        cidx = lax.axis_index("c")
        sidx = lax.axis_index("s")
        base = sidx * per_tile

        pltpu.sync_copy(z_h.at[pl.ds(base, per_tile)],
                        agg.at[pl.ds(base, per_tile)])
        plsc.subcore_barrier()

        def body(x_ref, i_ref):
            pltpu.sync_copy(x_ref, agg.at[i_ref.at[0]], add=True)

        pltpu.emit_pipeline(
            body,
            grid=(e_tot // _CHUNK,),
            in_specs=[_row_blk_spec(), _idx_spec()],
            out_specs=[],
            core_axis_name=("c", "s"),
            dimension_semantics=(pltpu.PARALLEL,),
        )(p_h, i_h)

        plsc.subcore_barrier()

        @pl.loop(0, per_tile // _CHUNK)
        def _(j):
            off = base + j * _CHUNK
            pltpu.sync_copy(agg.at[pl.ds(off, _CHUNK)],
                            o_h.at[cidx, pl.ds(off, _CHUNK)])

    return k(p, idx, jnp.zeros((_N_ACC, LATENT), f32))


def kernel(node_features, mesh_edge_features, world_edge_features,
           mesh_senders, mesh_receivers, world_senders, world_receivers,
           params, *, interpret=False):
    n = node_features.shape[0]
    em = mesh_edge_features.shape[0]
    ew = world_edge_features.shape[0]
    span = _CHUNK * _NW
    em_pad = ((em + span - 1) // span) * span
    ew_pad = ((ew + span - 1) // span) * span
    e_tot = em_pad + ew_pad

    f32 = jnp.float32

    def pad_idx(ix, tot, fill):
        ix = ix.astype(jnp.int32)
        return jnp.pad(ix, (0, tot - ix.shape[0]), constant_values=fill)

    i_s = jnp.concatenate([pad_idx(mesh_senders, em_pad, 0),
                           pad_idx(world_senders, ew_pad, 0) + n]
                          ).reshape(1, e_tot)
    i_r = jnp.concatenate([pad_idx(mesh_receivers, em_pad, 0),
                           pad_idx(world_receivers, ew_pad, 0) + n]
                          ).reshape(1, e_tot)
    i_sc = jnp.concatenate([pad_idx(mesh_receivers, em_pad, n),
                            pad_idx(world_receivers, ew_pad, n)]
                           ).reshape(1, e_tot)

    def pad_cols(x, to):
        return jnp.pad(x, ((0, 0), (0, to - x.shape[1])))

    def fold_norm(p, mean, std):
        w1 = p["W1"] / std[:, None]
        b1 = p["b1"] - (mean / std) @ p["W1"]
        return w1, b1

    BN = 2000

    def enc(x, w1, b1, p):
        din = x.shape[1]
        dpad = 16
        xp = pad_cols(x.astype(f32), dpad)
        w1p = jnp.pad(w1, ((0, dpad - din), (0, 0)))
        pa = [w1p, b1.reshape(1, -1), p["W2"], p["b2"].reshape(1, -1),
              p["g"].reshape(1, -1), p["beta"].reshape(1, -1)]
        return _call_rows(_enc_body, x.shape[0], BN, [xp], [dpad], pa, LATENT,
                          interpret=interpret)

    pe = params["enc_node"]
    node = enc(node_features, pe["W1"], pe["b1"], pe)
    pm_ = params["enc_mesh"]
    w1m, b1m = fold_norm(pm_, params["mesh_norm_mean"], params["mesh_norm_std"])
    me = enc(mesh_edge_features, w1m, b1m, pm_)
    pw_ = params["enc_world"]
    w1w, b1w = fold_norm(pw_, params["world_norm_mean"], params["world_norm_std"])
    we = enc(world_edge_features, w1w, b1w, pw_)

    e_lat = jnp.concatenate([jnp.pad(me, ((0, em_pad - em), (0, 0))),
                             jnp.pad(we, ((0, ew_pad - ew), (0, 0)))])

    steps = params["proc_mesh"]["W1"].shape[0]
    BE = 4096
    mesh_blocks = em_pad // BE

    def split_w1(w1):
        return w1[:LATENT], w1[LATENT:2 * LATENT], w1[2 * LATENT:]

    for i in range(steps):
        pmesh = jax.tree_util.tree_map(lambda a: a[i], params["proc_mesh"])
        pworld = jax.tree_util.tree_map(lambda a: a[i], params["proc_world"])
        pnode = jax.tree_util.tree_map(lambda a: a[i], params["proc_node"])

        w1a_m, w1b_m, w1c_m = split_w1(pmesh["W1"])
        w1a_w, w1b_w, w1c_w = split_w1(pworld["W1"])
        w1a_n, w1b_n, w1c_n = split_w1(pnode["W1"])

        wstack = jnp.concatenate([w1b_m, w1c_m, w1b_w, w1c_w], axis=1)
        zs, zr = _call_proj(node, wstack, n, BN, interpret=interpret)

        gs, gr = _sc_gather_sum(zs.reshape(2 * n, LATENT),
                                zr.reshape(2 * n, LATENT), i_s, i_r, e_tot)

        stacked = [
            jnp.stack([w1a_m, w1a_w]),
            jnp.stack([pmesh["b1"], pworld["b1"]]).reshape(2, 1, LATENT),
            jnp.stack([pmesh["W2"], pworld["W2"]]),
            jnp.stack([pmesh["b2"], pworld["b2"]]).reshape(2, 1, LATENT),
            jnp.stack([pmesh["g"], pworld["g"]]).reshape(2, 1, LATENT),
            jnp.stack([pmesh["beta"], pworld["beta"]]).reshape(2, 1, LATENT),
            jnp.stack([w1b_n, w1c_n]),
        ]
        e_lat, p_proj = _call_edges(e_lat, gs, gr, stacked, e_tot, BE,
                                    mesh_blocks, interpret=interpret)

        part = _sc_segment_sum(p_proj, i_sc, e_tot)

        pa_n = [w1a_n, pnode["b1"].reshape(1, -1), pnode["W2"],
                pnode["b2"].reshape(1, -1), pnode["g"].reshape(1, -1),
                pnode["beta"].reshape(1, -1)]
        node = _call_rows(_node_body, n, BN, [node, part[0], part[1]],
                          [LATENT] * 3, pa_n, LATENT, interpret=interpret)

    pd = params["dec"]
    out_dim = pd["W2"].shape[1]
    w2p = pad_cols(pd["W2"], LATENT)
    b2p = jnp.pad(pd["b2"], (0, LATENT - out_dim))
    pa_d = [pd["W1"], pd["b1"].reshape(1, -1), w2p, b2p.reshape(1, -1)]
    out = _call_rows(_dec_body, n, BN, [node], [LATENT], pa_d, LATENT,
                     interpret=interpret)
    return out[:, :out_dim]
